# Initial kernel scaffold; baseline (speedup 1.0000x reference)
import jax
import jax.numpy as jnp
from jax.experimental import pallas as pl


def kernel(x, edge_index, edge_weight, batch, W1_rel, b1_rel, W1_root, W2_rel, b2_rel, W2_root, W3_rel, b3_rel, W3_root):
    raise NotImplementedError("write your pallas kernel here")


# speedup vs baseline: 3.2612x; 3.2612x over previous
import functools

import jax
import jax.numpy as jnp
from jax import lax
from jax.experimental import pallas as pl
from jax.experimental.pallas import tpu as pltpu
from jax.experimental.pallas import tpu_sc as plsc

NC = 2
NS = 16
NW = NC * NS
LANES = 16
CHUNK = 128


def _sc_agg(x, srcp, dstp, wp):
    N, D = x.shape
    _, K, C = srcp.shape
    assert D % LANES == 0
    STRIPE = 640
    LAST = N - STRIPE * (NS - 1)
    ZR = 80
    assert STRIPE % ZR == 0 and LAST % ZR == 0 and LAST > 0
    nfeat = D // LANES

    mesh = plsc.VectorSubcoreMesh(core_axis_name="c", subcore_axis_name="s")

    @functools.partial(
        pl.kernel,
        out_type=jax.ShapeDtypeStruct((NC, N, D), jnp.float32),
        mesh=mesh,
        scratch_types=[
            pltpu.VMEM((C,), jnp.int32),
            pltpu.VMEM((C,), jnp.int32),
            pltpu.VMEM((C,), jnp.float32),
            pltpu.VMEM((C, D), jnp.float32),
            pltpu.VMEM((ZR, D), jnp.float32),
            pltpu.VMEM_SHARED((N, D), jnp.float32),
            pltpu.SemaphoreType.DMA,
        ],
        compiler_params=pltpu.CompilerParams(needs_layout_passes=False),
    )
    def k(x_hbm, src_hbm, dst_hbm, w_hbm, out_hbm,
          src_v, dst_v, w_v, rows_v, zero_v, agg_sh, sem):
        c = lax.axis_index("c")
        s = lax.axis_index("s")
        wid = s * NC + c

        def zrow(i, _):
            for f in range(nfeat):
                zero_v[i, pl.ds(f * LANES, LANES)] = jnp.zeros(
                    (LANES,), jnp.float32)
            return 0
        lax.fori_loop(0, ZR, zrow, 0)
        base = pl.multiple_of(s * STRIPE, 8)

        @pl.when(s < NS - 1)
        def _():
            for z in range(STRIPE // ZR):
                pltpu.sync_copy(zero_v, agg_sh.at[pl.ds(base + z * ZR, ZR)])

        @pl.when(s == NS - 1)
        def _():
            for z in range(LAST // ZR):
                pltpu.sync_copy(zero_v, agg_sh.at[pl.ds(base + z * ZR, ZR)])
        plsc.subcore_barrier()

        def chunk(j, _):
            pltpu.sync_copy(src_hbm.at[wid, j], src_v)
            pltpu.sync_copy(dst_hbm.at[wid, j], dst_v)
            pltpu.sync_copy(w_hbm.at[wid, j], w_v)
            pltpu.async_copy(x_hbm.at[src_v], rows_v, sem).wait()

            def edge(e, _):
                wspl = plsc.load_gather(
                    w_v, [jnp.full((LANES,), e, jnp.int32)])
                for f in range(nfeat):
                    sl = pl.ds(f * LANES, LANES)
                    rows_v[e, sl] = rows_v[e, sl] * wspl
                return 0
            lax.fori_loop(0, C, edge, 0)

            pltpu.sync_copy(rows_v, agg_sh.at[dst_v], add=True)
            return 0
        lax.fori_loop(0, K, chunk, 0)

        plsc.subcore_barrier()

        @pl.when(s < NS - 1)
        def _():
            pltpu.sync_copy(agg_sh.at[pl.ds(base, STRIPE)],
                            out_hbm.at[c, pl.ds(base, STRIPE)])

        @pl.when(s == NS - 1)
        def _():
            pltpu.sync_copy(agg_sh.at[pl.ds(base, LAST)],
                            out_hbm.at[c, pl.ds(base, LAST)])

    return k(x, srcp, dstp, wp)


def _tc_layer(partials, x, wrel_t, wroot_t, b2d, relu):
    N, D = x.shape
    BN = 1000
    assert N % BN == 0

    def body(p_ref, x_ref, wr_ref, wt_ref, b_ref, o_ref):
        agg = p_ref[0] + p_ref[1]
        acc = jnp.dot(agg, wr_ref[...], preferred_element_type=jnp.float32)
        acc = acc + jnp.dot(x_ref[...], wt_ref[...],
                            preferred_element_type=jnp.float32)
        acc = acc + b_ref[...]
        if relu:
            acc = jnp.maximum(acc, 0.0)
        o_ref[...] = acc

    return pl.pallas_call(
        body,
        grid=(N // BN,),
        in_specs=[
            pl.BlockSpec((NC, BN, D), lambda i: (0, i, 0)),
            pl.BlockSpec((BN, D), lambda i: (i, 0)),
            pl.BlockSpec((D, D), lambda i: (0, 0)),
            pl.BlockSpec((D, D), lambda i: (0, 0)),
            pl.BlockSpec((1, D), lambda i: (0, 0)),
        ],
        out_specs=pl.BlockSpec((BN, D), lambda i: (i, 0)),
        out_shape=jax.ShapeDtypeStruct((N, D), jnp.float32),
    )(partials, x, wrel_t, wroot_t, b2d)


def kernel(x, edge_index, edge_weight, batch,
           W1_rel, b1_rel, W1_root, W2_rel, b2_rel, W2_root,
           W3_rel, b3_rel, W3_root):
    del batch
    N, D = x.shape
    E = edge_index.shape[1]

    per_w = -(-E // (NW * CHUNK)) * CHUNK
    e_pad = per_w * NW
    pad = e_pad - E
    src = jnp.concatenate(
        [edge_index[0], jnp.zeros((pad,), jnp.int32)]).reshape(NW, -1, CHUNK)
    dst = jnp.concatenate(
        [edge_index[1], jnp.zeros((pad,), jnp.int32)]).reshape(NW, -1, CHUNK)
    w = jnp.concatenate(
        [edge_weight, jnp.zeros((pad,), jnp.float32)]).reshape(NW, -1, CHUNK)

    h = x
    layers = [
        (W1_rel, b1_rel, W1_root, True),
        (W2_rel, b2_rel, W2_root, True),
        (W3_rel, b3_rel, W3_root, False),
    ]
    for W_rel, b_rel, W_root, relu in layers:
        partials = _sc_agg(h, src, dst, w)
        h = _tc_layer(partials, h, W_rel.T, W_root.T, b_rel[None, :], relu)
    return h
